# baseline (speedup 1.0000x reference)
import functools

import jax
import jax.numpy as jnp
import numpy as np
from jax import lax
from jax.experimental import pallas as pl
from jax.experimental.pallas import tpu as pltpu

_EPS = 1e-5


def _shift_lanes(s, off):
    if off == 0:
        return s
    if off > 0:
        return jnp.concatenate(
            [s[:, off:], jnp.zeros((s.shape[0], off), s.dtype)], axis=1)
    return jnp.concatenate(
        [jnp.zeros((s.shape[0], -off), s.dtype), s[:, :off]], axis=1)


def _up_kernel(c_ref, hp_ref, w_ref, r_ref, y_ref, s_ref, *, D, H, W):
    d0 = (2 * pl.program_id(0)) % D
    s0 = c_ref[0]
    s1 = c_ref[1]
    s2 = jnp.where(d0 < D - 2, hp_ref[0], jnp.zeros_like(hp_ref[0]))
    slabs = (s0, s1, s2)
    lane = lax.broadcasted_iota(jnp.int32, (1, H * W), 1)
    h = lane // W
    w = lane % W
    rows = []
    for sd in (0, 1):
        for sh in (0, 1):
            for sw in (0, 1):
                off = sh * W + sw
                valid = (h + sh < H) & (w + sw < W)
                parts = []
                for j in (0, 1):
                    t = _shift_lanes(slabs[j + sd], off)
                    parts.append(jnp.where(valid, t, jnp.zeros_like(t)))
                rows.append(jnp.concatenate(parts, axis=1))
    xm = jnp.concatenate(rows, axis=0)
    y = jnp.dot(w_ref[...], xm, preferred_element_type=jnp.float32)
    yb = y.astype(jnp.bfloat16)
    Cout = y.shape[0] // 8
    for j in range(2):
        yj = yb[:, j * H * W:(j + 1) * H * W]
        ycat = jnp.concatenate(
            [yj[p * 2 * Cout:(p + 1) * 2 * Cout] for p in range(4)],
            axis=1)
        o = jnp.dot(ycat, r_ref[...], preferred_element_type=jnp.float32)
        ob = o.astype(jnp.bfloat16)
        y_ref[2 * j] = ob[:Cout]
        y_ref[2 * j + 1] = ob[Cout:]
    sums = jnp.sum(y, axis=1, keepdims=True)
    sqs = jnp.sum(y * y, axis=1, keepdims=True)
    s_ref[...] = jnp.concatenate([sums, sqs], axis=1)[None]


def _phase_weight(w_up):
    sel = np.zeros((2, 2, 3), np.float32)
    sel[0, 0, 1] = 1.0
    sel[1, 0, 2] = 1.0
    sel[1, 1, 0] = 1.0
    sel = jnp.asarray(sel)
    w8 = jnp.einsum('PSa,QTb,RUc,ioabc->QRPoSTUi', sel, sel, sel,
                    w_up.astype(jnp.float32))
    Cout, Cin = w_up.shape[1], w_up.shape[0]
    return w8.reshape(8 * Cout, 8 * Cin)


def _shift_matrices(Ho, Wo):
    n = Ho * Wo
    sm = np.zeros((9, n, n), np.float32)
    for kh in range(3):
        for kw in range(3):
            t = kh * 3 + kw
            for l in range(n):
                h, w = divmod(l, Wo)
                hh, ww = h + kh - 1, w + kw - 1
                if 0 <= hh < Ho and 0 <= ww < Wo:
                    sm[t, hh * Wo + ww, l] = 1.0
    return jnp.asarray(sm)


def _toeplitz_weight(w_res, TD):
    C = w_res.shape[0]
    wt9 = jnp.transpose(w_res, (2, 3, 4, 0, 1))
    wt = jnp.zeros((9, TD * C, (TD + 2) * C), jnp.float32)
    for kh in range(3):
        for kw in range(3):
            for j in range(TD):
                for kd in range(3):
                    wt = wt.at[kh * 3 + kw, j * C:(j + 1) * C,
                               (j + kd) * C:(j + kd + 1) * C].set(
                                   wt9[kd, kh, kw])
    return wt


def _deint_matrix(H, W):
    n = 4 * H * W
    r = np.zeros((n, n), np.float32)
    for ph in range(2):
        for pw in range(2):
            for h in range(H):
                for w in range(W):
                    src = (ph * 2 + pw) * H * W + h * W + w
                    dst = (2 * h + ph) * 2 * W + 2 * w + pw
                    r[src, dst] = 1.0
    return jnp.asarray(r)


def _res_kernel(hm_ref, c_ref, hp_ref, wt_ref, sm_ref, z_ref, s_ref,
                *, TD, Do, Ho, Wo):
    do0 = (TD * pl.program_id(0)) % Do
    s_lo = jnp.where(do0 > 0, hm_ref[0], jnp.zeros_like(hm_ref[0]))
    s_hi = jnp.where(do0 < Do - TD, hp_ref[0], jnp.zeros_like(hp_ref[0]))
    sfull = jnp.concatenate(
        (s_lo,) + tuple(c_ref[j] for j in range(TD)) + (s_hi,),
        axis=0)
    z = None
    for t in range(9):
        sh = jnp.dot(sfull, sm_ref[t], preferred_element_type=jnp.float32)
        p = jnp.dot(wt_ref[t], sh.astype(jnp.bfloat16),
                    preferred_element_type=jnp.float32)
        z = p if z is None else z + p
    C = z.shape[0] // TD
    zb = z.astype(jnp.bfloat16)
    for j in range(TD):
        z_ref[j] = zb[j * C:(j + 1) * C]
    sums = jnp.sum(z, axis=1, keepdims=True)
    sqs = jnp.sum(z * z, axis=1, keepdims=True)
    st = jnp.concatenate([sums, sqs], axis=1)
    s_ref[...] = st.reshape(TD, C, 2).sum(axis=0)[None]


def _final_kernel(z_ref, y_ref, sc2_ref, sh2_ref, al2_ref, o_ref):
    a2 = al2_ref[0, 0]
    for j in range(z_ref.shape[0]):
        t2 = (z_ref[j].astype(jnp.float32) * sc2_ref[...] + sh2_ref[...])
        o_ref[j] = (jnp.where(t2 > 0, t2, a2 * t2)
                    + y_ref[j].astype(jnp.float32))


def _finalize_bn(s, count, gamma, beta):
    mean = s[:, 0] / count
    var = jnp.maximum(s[:, 1] / count - mean * mean, 0.0)
    scale = gamma.astype(jnp.float32) * lax.rsqrt(var + _EPS)
    shift = beta.astype(jnp.float32) - mean * scale
    return scale.reshape(-1, 1), shift.reshape(-1, 1)


def kernel(x, w_up, b_up, gamma1, beta1, alpha1,
           w_res, b_res, gamma2, beta2, alpha2):
    N, Cin, D, H, W = x.shape
    Cout = w_up.shape[1]
    Do, Ho, Wo = 2 * D, 2 * H, 2 * W
    HW, HWo = H * W, Ho * Wo
    count = N * Do * HWo

    x_t = (x.astype(jnp.bfloat16)
           .transpose(0, 2, 1, 3, 4).reshape(N * D, Cin, HW))
    w8 = _phase_weight(w_up).astype(jnp.bfloat16)
    nd = N * D
    ga = nd // 2
    y_d, st1 = pl.pallas_call(
        functools.partial(_up_kernel, D=D, H=H, W=W),
        out_shape=(jax.ShapeDtypeStruct((N * Do, Cout, HWo), jnp.bfloat16),
                   jax.ShapeDtypeStruct((ga, 8 * Cout, 2), jnp.float32)),
        grid=(ga,),
        in_specs=[
            pl.BlockSpec((2, Cin, HW), lambda i: (i, 0, 0)),
            pl.BlockSpec((1, Cin, HW),
                         lambda i: (jnp.minimum(2 * i + 2, nd - 1), 0, 0)),
            pl.BlockSpec((8 * Cout, 8 * Cin), lambda i: (0, 0)),
            pl.BlockSpec((4 * HW, 4 * HW), lambda i: (0, 0)),
        ],
        out_specs=(pl.BlockSpec((4, Cout, HWo), lambda i: (i, 0, 0)),
                   pl.BlockSpec((1, 8 * Cout, 2), lambda i: (i, 0, 0))),
        compiler_params=pltpu.CompilerParams(dimension_semantics=("parallel",)),
    )(x_t, x_t, w8, _deint_matrix(H, W).astype(jnp.bfloat16))

    s1 = st1.sum(axis=0).reshape(8, Cout, 2).sum(axis=0)
    sc1, sh1 = _finalize_bn(s1, count, gamma1, beta1)

    t1 = y_d.astype(jnp.float32) * sc1[None] + sh1[None]
    y_act = jnp.where(t1 > 0, t1, alpha1 * t1).astype(jnp.bfloat16)

    nrow = N * Do
    TD = 4
    gb = nrow // TD
    wt = _toeplitz_weight(w_res, TD).astype(jnp.bfloat16)
    sm = _shift_matrices(Ho, Wo).astype(jnp.bfloat16)
    z, st2 = pl.pallas_call(
        functools.partial(_res_kernel, TD=TD, Do=Do, Ho=Ho, Wo=Wo),
        out_shape=(jax.ShapeDtypeStruct((nrow, Cout, HWo), jnp.bfloat16),
                   jax.ShapeDtypeStruct((gb, Cout, 2), jnp.float32)),
        grid=(gb,),
        in_specs=[
            pl.BlockSpec((1, Cout, HWo),
                         lambda i: (jnp.maximum(TD * i - 1, 0), 0, 0)),
            pl.BlockSpec((TD, Cout, HWo), lambda i: (i, 0, 0)),
            pl.BlockSpec((1, Cout, HWo),
                         lambda i: (jnp.minimum(TD * i + TD, nrow - 1), 0, 0)),
            pl.BlockSpec((9, TD * Cout, (TD + 2) * Cout),
                         lambda i: (0, 0, 0)),
            pl.BlockSpec((9, HWo, HWo), lambda i: (0, 0, 0)),
        ],
        out_specs=(pl.BlockSpec((TD, Cout, HWo), lambda i: (i, 0, 0)),
                   pl.BlockSpec((1, Cout, 2), lambda i: (i, 0, 0))),
        compiler_params=pltpu.CompilerParams(dimension_semantics=("parallel",)),
    )(y_act, y_act, y_act, wt, sm)

    sc2, sh2 = _finalize_bn(st2.sum(axis=0), count, gamma2, beta2)

    out_s = pl.pallas_call(
        _final_kernel,
        out_shape=jax.ShapeDtypeStruct((nrow, Cout, HWo), jnp.float32),
        grid=(gb,),
        in_specs=[
            pl.BlockSpec((TD, Cout, HWo), lambda i: (i, 0, 0)),
            pl.BlockSpec((TD, Cout, HWo), lambda i: (i, 0, 0)),
            pl.BlockSpec((Cout, 1), lambda i: (0, 0)),
            pl.BlockSpec((Cout, 1), lambda i: (0, 0)),
            pl.BlockSpec((1, 1), lambda i: (0, 0)),
        ],
        out_specs=pl.BlockSpec((TD, Cout, HWo), lambda i: (i, 0, 0)),
        compiler_params=pltpu.CompilerParams(dimension_semantics=("parallel",)),
    )(z, y_act, sc2, sh2, jnp.full((1, 1), alpha2, jnp.float32))

    return (out_s.reshape(N, Do, Cout, HWo).transpose(0, 2, 1, 3)
            .reshape(N, Cout, Do, Ho, Wo))


# speedup vs baseline: 2.2373x; 2.2373x over previous
import functools

import jax
import jax.numpy as jnp
import numpy as np
from jax import lax
from jax.experimental import pallas as pl
from jax.experimental.pallas import tpu as pltpu

_EPS = 1e-5


def _shift_lanes(s, off):
    if off == 0:
        return s
    if off > 0:
        return jnp.concatenate(
            [s[:, off:], jnp.zeros((s.shape[0], off), s.dtype)], axis=1)
    return jnp.concatenate(
        [jnp.zeros((s.shape[0], -off), s.dtype), s[:, :off]], axis=1)


def _up_kernel(c_ref, hp_ref, w_ref, r_ref, y_ref, s_ref, *, D, H, W):
    d0 = (2 * pl.program_id(0)) % D
    s0 = c_ref[0]
    s1 = c_ref[1]
    s2 = jnp.where(d0 < D - 2, hp_ref[0], jnp.zeros_like(hp_ref[0]))
    slabs = (s0, s1, s2)
    lane = lax.broadcasted_iota(jnp.int32, (1, H * W), 1)
    h = lane // W
    w = lane % W
    rows = []
    for sd in (0, 1):
        for sh in (0, 1):
            for sw in (0, 1):
                off = sh * W + sw
                valid = (h + sh < H) & (w + sw < W)
                parts = []
                for j in (0, 1):
                    t = _shift_lanes(slabs[j + sd], off)
                    parts.append(jnp.where(valid, t, jnp.zeros_like(t)))
                rows.append(jnp.concatenate(parts, axis=1))
    xm = jnp.concatenate(rows, axis=0)
    y = jnp.dot(w_ref[...], xm, preferred_element_type=jnp.float32)
    yb = y.astype(jnp.bfloat16)
    Cout = y.shape[0] // 8
    for j in range(2):
        yj = yb[:, j * H * W:(j + 1) * H * W]
        ycat = jnp.concatenate(
            [yj[p * 2 * Cout:(p + 1) * 2 * Cout] for p in range(4)],
            axis=1)
        o = jnp.dot(ycat, r_ref[...], preferred_element_type=jnp.float32)
        ob = o.astype(jnp.bfloat16)
        y_ref[2 * j] = ob[:Cout]
        y_ref[2 * j + 1] = ob[Cout:]
    sums = jnp.sum(y, axis=1, keepdims=True)
    sqs = jnp.sum(y * y, axis=1, keepdims=True)
    s_ref[...] = jnp.concatenate([sums, sqs], axis=1)[None]


def _phase_weight(w_up):
    sel = np.zeros((2, 2, 3), np.float32)
    sel[0, 0, 1] = 1.0
    sel[1, 0, 2] = 1.0
    sel[1, 1, 0] = 1.0
    sel = jnp.asarray(sel)
    w8 = jnp.einsum('PSa,QTb,RUc,ioabc->QRPoSTUi', sel, sel, sel,
                    w_up.astype(jnp.float32))
    Cout, Cin = w_up.shape[1], w_up.shape[0]
    return w8.reshape(8 * Cout, 8 * Cin)


def _deint_matrix(H, W):
    n = 4 * H * W
    r = np.zeros((n, n), np.float32)
    for ph in range(2):
        for pw in range(2):
            for h in range(H):
                for w in range(W):
                    src = (ph * 2 + pw) * H * W + h * W + w
                    dst = (2 * h + ph) * 2 * W + 2 * w + pw
                    r[src, dst] = 1.0
    return jnp.asarray(r)


def _res_kernel(hm_ref, c_ref, hp_ref, w_ref, z_ref, s_ref, *, TD, Do, Ho, Wo):
    do0 = (TD * pl.program_id(0)) % Do
    s_lo = jnp.where(do0 > 0, hm_ref[0], jnp.zeros_like(hm_ref[0]))
    s_hi = jnp.where(do0 < Do - TD, hp_ref[0], jnp.zeros_like(hp_ref[0]))
    slabs = (s_lo,) + tuple(c_ref[j] for j in range(TD)) + (s_hi,)

    lane = lax.broadcasted_iota(jnp.int32, (1, Ho * Wo), 1)
    h = lane // Wo
    w = lane % Wo
    rows = [None] * 27
    for kh in (0, 1, 2):
        for kw in (0, 1, 2):
            off = (kh - 1) * Wo + (kw - 1)
            valid = ((h + kh - 1 >= 0) & (h + kh - 1 < Ho)
                     & (w + kw - 1 >= 0) & (w + kw - 1 < Wo))
            shifted = [jnp.where(valid, _shift_lanes(s, off),
                                 jnp.zeros_like(s)) for s in slabs]
            for kd in (0, 1, 2):
                rows[(kd * 3 + kh) * 3 + kw] = jnp.concatenate(
                    shifted[kd:kd + TD], axis=1)
    xm = jnp.concatenate(rows, axis=0)
    z = jnp.dot(w_ref[...], xm, preferred_element_type=jnp.float32)
    zb = z.astype(jnp.bfloat16)
    for j in range(TD):
        z_ref[j] = zb[:, j * Ho * Wo:(j + 1) * Ho * Wo]
    sums = jnp.sum(z, axis=1, keepdims=True)
    sqs = jnp.sum(z * z, axis=1, keepdims=True)
    s_ref[...] = jnp.concatenate([sums, sqs], axis=1)[None]


def _final_kernel(z_ref, y_ref, sc2_ref, sh2_ref, al2_ref, o_ref):
    a2 = al2_ref[0, 0]
    for j in range(z_ref.shape[0]):
        t2 = (z_ref[j].astype(jnp.float32) * sc2_ref[...] + sh2_ref[...])
        o_ref[j] = (jnp.where(t2 > 0, t2, a2 * t2)
                    + y_ref[j].astype(jnp.float32))


def _finalize_bn(s, count, gamma, beta):
    mean = s[:, 0] / count
    var = jnp.maximum(s[:, 1] / count - mean * mean, 0.0)
    scale = gamma.astype(jnp.float32) * lax.rsqrt(var + _EPS)
    shift = beta.astype(jnp.float32) - mean * scale
    return scale.reshape(-1, 1), shift.reshape(-1, 1)


def kernel(x, w_up, b_up, gamma1, beta1, alpha1,
           w_res, b_res, gamma2, beta2, alpha2):
    N, Cin, D, H, W = x.shape
    Cout = w_up.shape[1]
    Do, Ho, Wo = 2 * D, 2 * H, 2 * W
    HW, HWo = H * W, Ho * Wo
    count = N * Do * HWo

    x_t = (x.astype(jnp.bfloat16)
           .transpose(0, 2, 1, 3, 4).reshape(N * D, Cin, HW))
    w8 = _phase_weight(w_up).astype(jnp.bfloat16)
    nd = N * D
    ga = nd // 2
    y_d, st1 = pl.pallas_call(
        functools.partial(_up_kernel, D=D, H=H, W=W),
        out_shape=(jax.ShapeDtypeStruct((N * Do, Cout, HWo), jnp.bfloat16),
                   jax.ShapeDtypeStruct((ga, 8 * Cout, 2), jnp.float32)),
        grid=(ga,),
        in_specs=[
            pl.BlockSpec((2, Cin, HW), lambda i: (i, 0, 0)),
            pl.BlockSpec((1, Cin, HW),
                         lambda i: (jnp.minimum(2 * i + 2, nd - 1), 0, 0)),
            pl.BlockSpec((8 * Cout, 8 * Cin), lambda i: (0, 0)),
            pl.BlockSpec((4 * HW, 4 * HW), lambda i: (0, 0)),
        ],
        out_specs=(pl.BlockSpec((4, Cout, HWo), lambda i: (i, 0, 0)),
                   pl.BlockSpec((1, 8 * Cout, 2), lambda i: (i, 0, 0))),
        compiler_params=pltpu.CompilerParams(dimension_semantics=("parallel",)),
    )(x_t, x_t, w8, _deint_matrix(H, W).astype(jnp.bfloat16))

    s1 = st1.sum(axis=0).reshape(8, Cout, 2).sum(axis=0)
    sc1, sh1 = _finalize_bn(s1, count, gamma1, beta1)

    t1 = y_d.astype(jnp.float32) * sc1[None] + sh1[None]
    y_act = jnp.where(t1 > 0, t1, alpha1 * t1).astype(jnp.bfloat16)

    w_r = jnp.transpose(w_res, (0, 2, 3, 4, 1)).reshape(Cout, 27 * Cout)
    w_r = w_r.astype(jnp.bfloat16)
    nrow = N * Do
    TD = 8
    gb = nrow // TD
    z, st2 = pl.pallas_call(
        functools.partial(_res_kernel, TD=TD, Do=Do, Ho=Ho, Wo=Wo),
        out_shape=(jax.ShapeDtypeStruct((nrow, Cout, HWo), jnp.bfloat16),
                   jax.ShapeDtypeStruct((gb, Cout, 2), jnp.float32)),
        grid=(gb,),
        in_specs=[
            pl.BlockSpec((1, Cout, HWo),
                         lambda i: (jnp.maximum(TD * i - 1, 0), 0, 0)),
            pl.BlockSpec((TD, Cout, HWo), lambda i: (i, 0, 0)),
            pl.BlockSpec((1, Cout, HWo),
                         lambda i: (jnp.minimum(TD * i + TD, nrow - 1), 0, 0)),
            pl.BlockSpec((Cout, 27 * Cout), lambda i: (0, 0)),
        ],
        out_specs=(pl.BlockSpec((TD, Cout, HWo), lambda i: (i, 0, 0)),
                   pl.BlockSpec((1, Cout, 2), lambda i: (i, 0, 0))),
        compiler_params=pltpu.CompilerParams(dimension_semantics=("parallel",)),
    )(y_act, y_act, y_act, w_r)

    sc2, sh2 = _finalize_bn(st2.sum(axis=0), count, gamma2, beta2)

    out_s = pl.pallas_call(
        _final_kernel,
        out_shape=jax.ShapeDtypeStruct((nrow, Cout, HWo), jnp.float32),
        grid=(gb,),
        in_specs=[
            pl.BlockSpec((TD, Cout, HWo), lambda i: (i, 0, 0)),
            pl.BlockSpec((TD, Cout, HWo), lambda i: (i, 0, 0)),
            pl.BlockSpec((Cout, 1), lambda i: (0, 0)),
            pl.BlockSpec((Cout, 1), lambda i: (0, 0)),
            pl.BlockSpec((1, 1), lambda i: (0, 0)),
        ],
        out_specs=pl.BlockSpec((TD, Cout, HWo), lambda i: (i, 0, 0)),
        compiler_params=pltpu.CompilerParams(dimension_semantics=("parallel",)),
    )(z, y_act, sc2, sh2, jnp.full((1, 1), alpha2, jnp.float32))

    return (out_s.reshape(N, Do, Cout, HWo).transpose(0, 2, 1, 3)
            .reshape(N, Cout, Do, Ho, Wo))


# speedup vs baseline: 2.3087x; 1.0319x over previous
import functools

import jax
import jax.numpy as jnp
import numpy as np
from jax import lax
from jax.experimental import pallas as pl
from jax.experimental.pallas import tpu as pltpu

_EPS = 1e-5


def _shift_lanes(s, off):
    if off == 0:
        return s
    if off > 0:
        return jnp.concatenate(
            [s[:, off:], jnp.zeros((s.shape[0], off), s.dtype)], axis=1)
    return jnp.concatenate(
        [jnp.zeros((s.shape[0], -off), s.dtype), s[:, :off]], axis=1)


def _up_kernel(c_ref, hp_ref, w_ref, r_ref, y_ref, s_ref, *, TA, D, H, W):
    d0 = (TA * pl.program_id(0)) % D
    slabs = [c_ref[j] for j in range(TA)]
    slabs.append(jnp.where(d0 < D - TA, hp_ref[0], jnp.zeros_like(hp_ref[0])))
    lane = lax.broadcasted_iota(jnp.int32, (1, H * W), 1)
    h = lane // W
    w = lane % W
    rows = []
    for sd in (0, 1):
        for sh in (0, 1):
            for sw in (0, 1):
                off = sh * W + sw
                valid = (h + sh < H) & (w + sw < W)
                parts = []
                for j in range(TA):
                    t = _shift_lanes(slabs[j + sd], off)
                    parts.append(jnp.where(valid, t, jnp.zeros_like(t)))
                rows.append(jnp.concatenate(parts, axis=1))
    xm = jnp.concatenate(rows, axis=0)
    y = jnp.dot(w_ref[...], xm, preferred_element_type=jnp.float32)
    yb = y.astype(jnp.bfloat16)
    Cout = y.shape[0] // 8
    for j in range(TA):
        yj = yb[:, j * H * W:(j + 1) * H * W]
        ycat = jnp.concatenate(
            [yj[p * 2 * Cout:(p + 1) * 2 * Cout] for p in range(4)],
            axis=1)
        o = jnp.dot(ycat, r_ref[...], preferred_element_type=jnp.float32)
        ob = o.astype(jnp.bfloat16)
        y_ref[2 * j] = ob[:Cout]
        y_ref[2 * j + 1] = ob[Cout:]
    sums = jnp.sum(y, axis=1, keepdims=True)
    sqs = jnp.sum(y * y, axis=1, keepdims=True)
    s_ref[...] = jnp.concatenate([sums, sqs], axis=1)[None]


def _phase_weight(w_up):
    sel = np.zeros((2, 2, 3), np.float32)
    sel[0, 0, 1] = 1.0
    sel[1, 0, 2] = 1.0
    sel[1, 1, 0] = 1.0
    sel = jnp.asarray(sel)
    w8 = jnp.einsum('PSa,QTb,RUc,ioabc->QRPoSTUi', sel, sel, sel,
                    w_up.astype(jnp.float32))
    Cout, Cin = w_up.shape[1], w_up.shape[0]
    return w8.reshape(8 * Cout, 8 * Cin)


def _deint_matrix(H, W):
    n = 4 * H * W
    r = np.zeros((n, n), np.float32)
    for ph in range(2):
        for pw in range(2):
            for h in range(H):
                for w in range(W):
                    src = (ph * 2 + pw) * H * W + h * W + w
                    dst = (2 * h + ph) * 2 * W + 2 * w + pw
                    r[src, dst] = 1.0
    return jnp.asarray(r)


def _res_kernel(hm_ref, c_ref, hp_ref, w_ref, z_ref, s_ref, *, TD, Do, Ho, Wo):
    do0 = (TD * pl.program_id(0)) % Do
    s_lo = jnp.where(do0 > 0, hm_ref[0], jnp.zeros_like(hm_ref[0]))
    s_hi = jnp.where(do0 < Do - TD, hp_ref[0], jnp.zeros_like(hp_ref[0]))
    slabs = (s_lo,) + tuple(c_ref[j] for j in range(TD)) + (s_hi,)

    lane = lax.broadcasted_iota(jnp.int32, (1, Ho * Wo), 1)
    h = lane // Wo
    w = lane % Wo
    rows = [None] * 27
    for kh in (0, 1, 2):
        for kw in (0, 1, 2):
            off = (kh - 1) * Wo + (kw - 1)
            valid = ((h + kh - 1 >= 0) & (h + kh - 1 < Ho)
                     & (w + kw - 1 >= 0) & (w + kw - 1 < Wo))
            shifted = [jnp.where(valid, _shift_lanes(s, off),
                                 jnp.zeros_like(s)) for s in slabs]
            for kd in (0, 1, 2):
                rows[(kd * 3 + kh) * 3 + kw] = jnp.concatenate(
                    shifted[kd:kd + TD], axis=1)
    xm = jnp.concatenate(rows, axis=0)
    z = jnp.dot(w_ref[...], xm, preferred_element_type=jnp.float32)
    zb = z.astype(jnp.bfloat16)
    for j in range(TD):
        z_ref[j] = zb[:, j * Ho * Wo:(j + 1) * Ho * Wo]
    sums = jnp.sum(z, axis=1, keepdims=True)
    sqs = jnp.sum(z * z, axis=1, keepdims=True)
    s_ref[...] = jnp.concatenate([sums, sqs], axis=1)[None]


def _final_kernel(z_ref, y_ref, sc2_ref, sh2_ref, al2_ref, o_ref):
    a2 = al2_ref[0, 0]
    for j in range(z_ref.shape[0]):
        t2 = (z_ref[j].astype(jnp.float32) * sc2_ref[...] + sh2_ref[...])
        o_ref[j] = (jnp.where(t2 > 0, t2, a2 * t2)
                    + y_ref[j].astype(jnp.float32))


def _finalize_bn(s, count, gamma, beta):
    mean = s[:, 0] / count
    var = jnp.maximum(s[:, 1] / count - mean * mean, 0.0)
    scale = gamma.astype(jnp.float32) * lax.rsqrt(var + _EPS)
    shift = beta.astype(jnp.float32) - mean * scale
    return scale.reshape(-1, 1), shift.reshape(-1, 1)


def kernel(x, w_up, b_up, gamma1, beta1, alpha1,
           w_res, b_res, gamma2, beta2, alpha2):
    N, Cin, D, H, W = x.shape
    Cout = w_up.shape[1]
    Do, Ho, Wo = 2 * D, 2 * H, 2 * W
    HW, HWo = H * W, Ho * Wo
    count = N * Do * HWo

    x_t = (x.astype(jnp.bfloat16)
           .transpose(0, 2, 1, 3, 4).reshape(N * D, Cin, HW))
    w8 = _phase_weight(w_up).astype(jnp.bfloat16)
    nd = N * D
    TA = 4
    ga = nd // TA
    y_d, st1 = pl.pallas_call(
        functools.partial(_up_kernel, TA=TA, D=D, H=H, W=W),
        out_shape=(jax.ShapeDtypeStruct((N * Do, Cout, HWo), jnp.bfloat16),
                   jax.ShapeDtypeStruct((ga, 8 * Cout, 2), jnp.float32)),
        grid=(ga,),
        in_specs=[
            pl.BlockSpec((TA, Cin, HW), lambda i: (i, 0, 0)),
            pl.BlockSpec((1, Cin, HW),
                         lambda i: (jnp.minimum(TA * i + TA, nd - 1), 0, 0)),
            pl.BlockSpec((8 * Cout, 8 * Cin), lambda i: (0, 0)),
            pl.BlockSpec((4 * HW, 4 * HW), lambda i: (0, 0)),
        ],
        out_specs=(pl.BlockSpec((2 * TA, Cout, HWo), lambda i: (i, 0, 0)),
                   pl.BlockSpec((1, 8 * Cout, 2), lambda i: (i, 0, 0))),
        compiler_params=pltpu.CompilerParams(dimension_semantics=("parallel",)),
    )(x_t, x_t, w8, _deint_matrix(H, W).astype(jnp.bfloat16))

    s1 = st1.sum(axis=0).reshape(8, Cout, 2).sum(axis=0)
    sc1, sh1 = _finalize_bn(s1, count, gamma1, beta1)

    t1 = y_d.astype(jnp.float32) * sc1[None] + sh1[None]
    y_act = jnp.where(t1 > 0, t1, alpha1 * t1).astype(jnp.bfloat16)

    w_r = jnp.transpose(w_res, (0, 2, 3, 4, 1)).reshape(Cout, 27 * Cout)
    w_r = w_r.astype(jnp.bfloat16)
    nrow = N * Do
    TD = 8
    gb = nrow // TD
    z, st2 = pl.pallas_call(
        functools.partial(_res_kernel, TD=TD, Do=Do, Ho=Ho, Wo=Wo),
        out_shape=(jax.ShapeDtypeStruct((nrow, Cout, HWo), jnp.bfloat16),
                   jax.ShapeDtypeStruct((gb, Cout, 2), jnp.float32)),
        grid=(gb,),
        in_specs=[
            pl.BlockSpec((1, Cout, HWo),
                         lambda i: (jnp.maximum(TD * i - 1, 0), 0, 0)),
            pl.BlockSpec((TD, Cout, HWo), lambda i: (i, 0, 0)),
            pl.BlockSpec((1, Cout, HWo),
                         lambda i: (jnp.minimum(TD * i + TD, nrow - 1), 0, 0)),
            pl.BlockSpec((Cout, 27 * Cout), lambda i: (0, 0)),
        ],
        out_specs=(pl.BlockSpec((TD, Cout, HWo), lambda i: (i, 0, 0)),
                   pl.BlockSpec((1, Cout, 2), lambda i: (i, 0, 0))),
        compiler_params=pltpu.CompilerParams(dimension_semantics=("parallel",)),
    )(y_act, y_act, y_act, w_r)

    sc2, sh2 = _finalize_bn(st2.sum(axis=0), count, gamma2, beta2)

    out_s = pl.pallas_call(
        _final_kernel,
        out_shape=jax.ShapeDtypeStruct((nrow, Cout, HWo), jnp.float32),
        grid=(gb,),
        in_specs=[
            pl.BlockSpec((TD, Cout, HWo), lambda i: (i, 0, 0)),
            pl.BlockSpec((TD, Cout, HWo), lambda i: (i, 0, 0)),
            pl.BlockSpec((Cout, 1), lambda i: (0, 0)),
            pl.BlockSpec((Cout, 1), lambda i: (0, 0)),
            pl.BlockSpec((1, 1), lambda i: (0, 0)),
        ],
        out_specs=pl.BlockSpec((TD, Cout, HWo), lambda i: (i, 0, 0)),
        compiler_params=pltpu.CompilerParams(dimension_semantics=("parallel",)),
    )(z, y_act, sc2, sh2, jnp.full((1, 1), alpha2, jnp.float32))

    return (out_s.reshape(N, Do, Cout, HWo).transpose(0, 2, 1, 3)
            .reshape(N, Cout, Do, Ho, Wo))
